# baseline (speedup 1.0000x reference)
import functools

import jax
import jax.numpy as jnp
from jax import lax
from jax.experimental import pallas as pl
from jax.experimental.pallas import tpu as pltpu

HID = 16
T = 5
START = 3
STOP = 4
NEG = -10000.0


def _bilstm_crf_fused(
    embs, wihf, whhf, bihf, bhhf, wihb, whhb, bihb, bhhb,
    w2t, bt, trans_ref, h0_ref, c0_ref,
    score_ref, path_ref,
    xf_ref, xb_ref, hst_ref, hrev_ref, bpo_ref,
):
    S = path_ref.shape[1]
    f32 = jnp.float32

    z16 = jnp.zeros((HID, HID), f32)
    z112 = jnp.zeros((HID, 112), f32)
    z96 = jnp.zeros((HID, 96), f32)

    def in_cat(w, fwd):
        blocks = []
        for j in range(4):
            blk = w[16 * j:16 * j + 16, :]
            blocks.extend([blk, z16] if fwd else [z16, blk])
        return jnp.concatenate(blocks, axis=0)

    wf_t = in_cat(wihf[...], True).T
    wb_t = in_cat(wihb[...], False).T

    rows = []
    for j in range(4):
        rows.append(jnp.concatenate([whhf[16 * j:16 * j + 16, :], z112], axis=1))
        rows.append(jnp.concatenate([z16, whhb[16 * j:16 * j + 16, :], z96], axis=1))
    whh_t = jnp.concatenate(rows, axis=0).T

    bf = bihf[...] + bhhf[...]
    bb = bihb[...] + bhhb[...]
    bias = jnp.concatenate(
        [x for j in range(4)
         for x in (bf[:, 16 * j:16 * j + 16], bb[:, 16 * j:16 * j + 16])],
        axis=1)

    w2t_t = w2t[...].T
    wa_t = jnp.concatenate([w2t_t[0:16, :], jnp.zeros((112, T), f32)],
                           axis=0)
    wb2_t = jnp.concatenate([jnp.zeros((16, T), f32), w2t_t[16:32, :],
                             jnp.zeros((96, T), f32)], axis=0)

    emb = embs[...]
    xf_ref[...] = jnp.dot(emb, wf_t, preferred_element_type=f32) + bias
    xb_ref[...] = jnp.dot(emb, wb_t, preferred_element_type=f32)

    z1_96 = jnp.zeros((1, 96), f32)
    h = jnp.concatenate([h0_ref[pl.ds(0, 1), :], h0_ref[pl.ds(1, 1), :], z1_96],
                        axis=1)
    c_st = jnp.concatenate([c0_ref[pl.ds(0, 1), :], c0_ref[pl.ds(1, 1), :], z1_96],
                           axis=1)

    for k in range(S):
        kr = S - 1 - k
        x = xf_ref[pl.ds(k, 1), :] + xb_ref[pl.ds(kr, 1), :]
        m = x + jnp.dot(h, whh_t, preferred_element_type=f32)
        sg = jax.nn.sigmoid(m)
        tg = jnp.tanh(m)
        c_st = pltpu.roll(sg, 96, 1) * c_st + sg * pltpu.roll(tg, 64, 1)
        h = pltpu.roll(sg, 32, 1) * jnp.tanh(c_st)
        hst_ref[pl.ds(k, 1), :] = h
        hrev_ref[pl.ds(kr, 1), :] = h

    feats = (jnp.dot(hst_ref[...], wa_t, preferred_element_type=f32)
             + jnp.dot(hrev_ref[...], wb2_t, preferred_element_type=f32)
             + bt[...])
    ft_t = feats.T

    lane_t = lax.broadcasted_iota(jnp.int32, (1, T), 1)
    lane2 = lax.broadcasted_iota(jnp.int32, (T, T), 1)
    sub2 = lax.broadcasted_iota(jnp.int32, (T, T), 0)
    lane_s = lax.broadcasted_iota(jnp.int32, (T, S), 1)
    trans = trans_ref[...]
    trans_t = trans.T

    fv_row = jnp.where(lane_t == START, 0.0, NEG)
    fv_col = None
    bp_cols = jnp.zeros((T, S), jnp.int32)
    for t in range(S):
        if t % 2 == 0:
            nvar = trans + fv_row
            best = jnp.max(nvar, axis=1, keepdims=True)
            bp = jnp.min(jnp.where(nvar == best, lane2, T), axis=1,
                         keepdims=True)
            bp_cols = jnp.where(lane_s == t, bp, bp_cols)
            fv_col = best + ft_t[:, t:t + 1]
        else:
            nvar = trans_t + fv_col
            best = jnp.max(nvar, axis=0, keepdims=True)
            bp = jnp.min(jnp.where(nvar == best, sub2, T), axis=0,
                         keepdims=True)
            bpo_ref[pl.ds(t, 1), :] = bp
            fv_row = best + feats[t:t + 1, :]

    terminal = fv_row + trans_ref[pl.ds(STOP, 1), :]
    path_score = jnp.max(terminal, axis=1, keepdims=True)
    score_ref[...] = path_score
    best_id = jnp.min(jnp.where(terminal == path_score, lane_t, T),
                      axis=1, keepdims=True)

    sub_t = lax.broadcasted_iota(jnp.int32, (T, 1), 0)
    iota_s = lax.broadcasted_iota(jnp.int32, (1, S), 1)
    path_row = jnp.where(iota_s == (S - 1), best_id, 0)
    cur = best_id
    for k in range(S - 1):
        t = S - 1 - k
        if t % 2 == 0:
            bp_t = bp_cols[:, t:t + 1]
            prev = jnp.sum(jnp.where(sub_t == cur, bp_t, 0), axis=0,
                           keepdims=True)
        else:
            bp_t = bpo_ref[pl.ds(t, 1), :]
            prev = jnp.sum(jnp.where(lane_t == cur, bp_t, 0), axis=1,
                           keepdims=True)
        path_row = jnp.where(iota_s == (t - 1), prev, path_row)
        cur = prev
    path_ref[...] = path_row


def kernel(sentence, embedding, w_ih_f, w_hh_f, b_ih_f, b_hh_f,
           w_ih_b, w_hh_b, b_ih_b, b_hh_b, w_h2t, b_h2t, transitions, h0, c0):
    S = sentence.shape[0]
    f32 = jnp.float32

    embs = jnp.concatenate(
        [lax.dynamic_slice(embedding, (sentence[k], 0), (1, embedding.shape[1]))
         for k in range(S)], axis=0)
    inputs = (
        embs,
        w_ih_f, w_hh_f, b_ih_f.reshape(1, 64), b_hh_f.reshape(1, 64),
        w_ih_b, w_hh_b, b_ih_b.reshape(1, 64), b_hh_b.reshape(1, 64),
        w_h2t, b_h2t.reshape(1, T), transitions,
        h0.reshape(2, HID), c0.reshape(2, HID),
    )

    def _vmem_spec(shape):
        nd = len(shape)
        return pl.BlockSpec(shape, lambda *_, _nd=nd: (0,) * _nd)

    in_specs = [_vmem_spec(x.shape) for x in inputs]

    score, path = pl.pallas_call(
        _bilstm_crf_fused,
        out_shape=(jax.ShapeDtypeStruct((1, 1), f32),
                   jax.ShapeDtypeStruct((1, S), jnp.int32)),
        grid_spec=pltpu.PrefetchScalarGridSpec(
            num_scalar_prefetch=0,
            grid=(1,),
            in_specs=in_specs,
            out_specs=[_vmem_spec((1, 1)), _vmem_spec((1, S))],
            scratch_shapes=[
                pltpu.VMEM((S, 128), f32),
                pltpu.VMEM((S, 128), f32),
                pltpu.VMEM((S, 128), f32),
                pltpu.VMEM((S, 128), f32),
                pltpu.VMEM((S, T), jnp.int32),
            ]),
        compiler_params=pltpu.CompilerParams(
            dimension_semantics=("arbitrary",)),
    )(*inputs)
    return score[0, 0], path[0, :]


# speedup vs baseline: 3.0862x; 3.0862x over previous
import jax
import jax.numpy as jnp
from jax import lax
from jax.experimental import pallas as pl
from jax.experimental.pallas import tpu as pltpu

HID = 16
EMB = 16
T = 5
START = 3
STOP = 4
NEG = -10000.0


def _bilstm_crf_fused(
    embs, wihf, whhf, bihf, bhhf, wihb, whhb, bihb, bhhb,
    w2t, bt, trans_ref, h0_ref, c0_ref,
    score_ref, path_ref,
    xf_ref, xb_ref, hst_ref, hrev_ref, bpc_ref, bpr_ref, term_ref,
    sm_bpc, sm_bpr, sm_term, sems,
):
    S = path_ref.shape[1]
    f32 = jnp.float32

    z16 = jnp.zeros((HID, HID), f32)
    z96w = jnp.zeros((96, EMB), f32)
    z96r = jnp.zeros((96, 2 * HID), f32)

    wf = wihf[...]
    wb = wihb[...]
    hf = whhf[...]
    hb = whhb[...]

    xrows_f, xrows_b, rrows = [], [], []
    for g in range(4):
        blk_f = wf[16 * g:16 * g + 16, :]
        blk_b = wb[16 * g:16 * g + 16, :]
        xrows_f += [blk_f, z16, z96w]
        xrows_b += [z16, blk_b, z96w]
        rrows += [jnp.concatenate([hf[16 * g:16 * g + 16, :], z16], axis=1),
                  jnp.concatenate([z16, hb[16 * g:16 * g + 16, :]], axis=1),
                  z96r]
    wx_f = jnp.concatenate(xrows_f, axis=0)
    wx_b = jnp.concatenate(xrows_b, axis=0)
    whh_t = jnp.concatenate(rrows, axis=0).T

    bf = bihf[...] + bhhf[...]
    bb = bihb[...] + bhhb[...]
    z1_96 = jnp.zeros((1, 96), f32)
    bias = jnp.concatenate(
        [x for g in range(4)
         for x in (bf[:, 16 * g:16 * g + 16], bb[:, 16 * g:16 * g + 16], z1_96)],
        axis=1)

    emb = embs[...]
    dn = (((1,), (1,)), ((), ()))
    xf_ref[...] = lax.dot_general(emb, wx_f, dn,
                                  preferred_element_type=f32) + bias
    xb_ref[...] = lax.dot_general(emb, wx_b, dn,
                                  preferred_element_type=f32)

    h = jnp.concatenate([h0_ref[0:1, :], h0_ref[1:2, :]], axis=1)
    c_st = jnp.concatenate([c0_ref[0:1, :], c0_ref[1:2, :]], axis=1)

    for k in range(S):
        kr = S - 1 - k
        x = xf_ref[pl.ds(k, 1), :] + xb_ref[pl.ds(kr, 1), :]
        m = x + jnp.dot(h, whh_t, preferred_element_type=f32)
        si = jax.nn.sigmoid(m[:, 0:32])
        sf = jax.nn.sigmoid(m[:, 128:160])
        tg = jnp.tanh(m[:, 256:288])
        so = jax.nn.sigmoid(m[:, 384:416])
        c_st = sf * c_st + si * tg
        h = so * jnp.tanh(c_st)
        hst_ref[pl.ds(k, 1), :] = h
        hrev_ref[pl.ds(kr, 1), :] = h

    a2 = jnp.concatenate([w2t[:, 0:16], jnp.zeros((T, HID), f32)], axis=1)
    b2 = jnp.concatenate([jnp.zeros((T, HID), f32), w2t[:, 16:32]], axis=1)
    feats = (lax.dot_general(hst_ref[...], a2, dn, preferred_element_type=f32)
             + lax.dot_general(hrev_ref[...], b2, dn, preferred_element_type=f32)
             + bt[...])
    ft_t = feats.T

    lane_t = lax.broadcasted_iota(jnp.int32, (1, T), 1)
    lane2 = lax.broadcasted_iota(jnp.int32, (T, T), 1)
    sub2 = lax.broadcasted_iota(jnp.int32, (T, T), 0)
    trans = trans_ref[...]
    trans_t = trans.T
    z55 = jnp.zeros((T, T), f32)
    ftreps = [ft_t[:, t:t + 1] + z55 for t in range(0, S, 2)]
    frows = [feats[t:t + 1, :] for t in range(1, S, 2)]

    fv_row = jnp.where(lane_t == START, 0.0, NEG)
    fv_col = None
    for t in range(S):
        if t % 2 == 0:
            nvar = trans + fv_row
            best = jnp.max(nvar, axis=1, keepdims=True)
            bp = jnp.min(jnp.where(nvar == best, lane2, T), axis=1,
                         keepdims=True)
            bpc_ref[:, t:t + 1] = bp
            fv_col = best + ftreps[t // 2]
        else:
            nvar = trans_t + fv_col
            best = jnp.max(nvar, axis=0, keepdims=True)
            bp = jnp.min(jnp.where(nvar == best, sub2, T), axis=0,
                         keepdims=True)
            bpr_ref[pl.ds(t, 1), :] = bp
            fv_row = best + frows[t // 2]

    terminal = fv_row + trans_ref[pl.ds(STOP, 1), :]
    score_ref[...] = jnp.max(terminal, axis=1, keepdims=True)
    term_ref[...] = terminal

    cps = [pltpu.make_async_copy(bpc_ref, sm_bpc, sems.at[0]),
           pltpu.make_async_copy(bpr_ref, sm_bpr, sems.at[1]),
           pltpu.make_async_copy(term_ref, sm_term, sems.at[2])]
    for cp in cps:
        cp.start()
    for cp in cps:
        cp.wait()

    best_v = sm_term[0, 0]
    best_i = jnp.int32(0)
    for j in range(1, T):
        better = sm_term[0, j] > best_v
        best_v = jnp.where(better, sm_term[0, j], best_v)
        best_i = jnp.where(better, jnp.int32(j), best_i)

    iota_s = lax.broadcasted_iota(jnp.int32, (1, S), 1)
    cur = best_i
    path_row = jnp.where(iota_s == (S - 1), cur, 0)
    for k in range(S - 1):
        t = S - 1 - k
        if t % 2 == 0:
            prev = sm_bpc[cur, t]
        else:
            prev = sm_bpr[t, cur]
        path_row = jnp.where(iota_s == (t - 1), prev, path_row)
        cur = prev
    path_ref[...] = path_row


def kernel(sentence, embedding, w_ih_f, w_hh_f, b_ih_f, b_hh_f,
           w_ih_b, w_hh_b, b_ih_b, b_hh_b, w_h2t, b_h2t, transitions, h0, c0):
    S = sentence.shape[0]
    f32 = jnp.float32

    inputs = (
        embedding[sentence],
        w_ih_f, w_hh_f, b_ih_f.reshape(1, 64), b_hh_f.reshape(1, 64),
        w_ih_b, w_hh_b, b_ih_b.reshape(1, 64), b_hh_b.reshape(1, 64),
        w_h2t, b_h2t.reshape(1, T), transitions,
        h0.reshape(2, HID), c0.reshape(2, HID),
    )

    def _vmem_spec(shape):
        nd = len(shape)
        return pl.BlockSpec(shape, lambda *_, _nd=nd: (0,) * _nd)

    score, path = pl.pallas_call(
        _bilstm_crf_fused,
        out_shape=(jax.ShapeDtypeStruct((1, 1), f32),
                   jax.ShapeDtypeStruct((1, S), jnp.int32)),
        grid_spec=pltpu.PrefetchScalarGridSpec(
            num_scalar_prefetch=0,
            grid=(1,),
            in_specs=[_vmem_spec(x.shape) for x in inputs],
            out_specs=[_vmem_spec((1, 1)), _vmem_spec((1, S))],
            scratch_shapes=[
                pltpu.VMEM((S, 512), f32),
                pltpu.VMEM((S, 512), f32),
                pltpu.VMEM((S, 2 * HID), f32),
                pltpu.VMEM((S, 2 * HID), f32),
                pltpu.VMEM((T, S), jnp.int32),
                pltpu.VMEM((S, T), jnp.int32),
                pltpu.VMEM((1, T), f32),
                pltpu.SMEM((T, S), jnp.int32),
                pltpu.SMEM((S, T), jnp.int32),
                pltpu.SMEM((1, T), f32),
                pltpu.SemaphoreType.DMA((3,)),
            ]),
        compiler_params=pltpu.CompilerParams(
            dimension_semantics=("arbitrary",)),
    )(*inputs)
    return score[0, 0], path[0, :]


# speedup vs baseline: 3.1064x; 1.0066x over previous
import jax
import jax.numpy as jnp
from jax import lax
from jax.experimental import pallas as pl
from jax.experimental.pallas import tpu as pltpu

HID = 16
EMB = 16
T = 5
START = 3
STOP = 4
NEG = -10000.0

_R_WIHF = 0
_R_WIHB = 64
_R_WHHF = 128
_R_WHHB = 192
_R_BF = 256
_R_BB = 260
_R_W2T = 264
_R_BT = 274
_R_TRANS = 275
_R_H0 = 280
_R_C0 = 282
_R_TOTAL = 284


def _bilstm_crf_fused(
    embs, wpk,
    score_ref, path_ref,
    xf_ref, xb_ref, hst_ref, hrev_ref, bpc_ref, bpr_ref, term_ref,
    sm_bpc, sm_bpr, sm_term, sems,
):
    S = path_ref.shape[1]
    f32 = jnp.float32

    z16 = jnp.zeros((HID, HID), f32)
    z96w = jnp.zeros((96, EMB), f32)
    z96r = jnp.zeros((96, 2 * HID), f32)

    xrows_f, xrows_b, rrows = [], [], []
    for g in range(4):
        blk_f = wpk[_R_WIHF + 16 * g:_R_WIHF + 16 * g + 16, :]
        blk_b = wpk[_R_WIHB + 16 * g:_R_WIHB + 16 * g + 16, :]
        hf = wpk[_R_WHHF + 16 * g:_R_WHHF + 16 * g + 16, :]
        hb = wpk[_R_WHHB + 16 * g:_R_WHHB + 16 * g + 16, :]
        xrows_f += [blk_f, z16, z96w]
        xrows_b += [z16, blk_b, z96w]
        rrows += [jnp.concatenate([hf, z16], axis=1),
                  jnp.concatenate([z16, hb], axis=1),
                  z96r]
    wx_f = jnp.concatenate(xrows_f, axis=0)
    wx_b = jnp.concatenate(xrows_b, axis=0)
    whh_t = jnp.concatenate(rrows, axis=0).T

    z1_96 = jnp.zeros((1, 96), f32)
    bias = jnp.concatenate(
        [x for g in range(4)
         for x in (wpk[_R_BF + g:_R_BF + g + 1, :],
                   wpk[_R_BB + g:_R_BB + g + 1, :], z1_96)],
        axis=1)

    w2t_f = wpk[_R_W2T + 0:_R_W2T + 5, :]
    w2t_b = wpk[_R_W2T + 5:_R_W2T + 10, :]
    zt16 = jnp.zeros((T, HID), f32)
    a2 = jnp.concatenate([w2t_f, zt16], axis=1)
    b2 = jnp.concatenate([zt16, w2t_b], axis=1)
    bt = wpk[_R_BT:_R_BT + 1, 0:T]
    trans = wpk[_R_TRANS:_R_TRANS + 5, 0:T]

    emb = embs[...]
    dn = (((1,), (1,)), ((), ()))
    xf_ref[...] = lax.dot_general(emb, wx_f, dn,
                                  preferred_element_type=f32) + bias
    xb_ref[...] = lax.dot_general(emb, wx_b, dn,
                                  preferred_element_type=f32)

    h = jnp.concatenate([wpk[_R_H0:_R_H0 + 1, :],
                         wpk[_R_H0 + 1:_R_H0 + 2, :]], axis=1)
    c_st = jnp.concatenate([wpk[_R_C0:_R_C0 + 1, :],
                            wpk[_R_C0 + 1:_R_C0 + 2, :]], axis=1)

    for k in range(S):
        kr = S - 1 - k
        x = xf_ref[pl.ds(k, 1), :] + xb_ref[pl.ds(kr, 1), :]
        m = x + jnp.dot(h, whh_t, preferred_element_type=f32)
        si = jax.nn.sigmoid(m[:, 0:32])
        sf = jax.nn.sigmoid(m[:, 128:160])
        tg = jnp.tanh(m[:, 256:288])
        so = jax.nn.sigmoid(m[:, 384:416])
        c_st = sf * c_st + si * tg
        h = so * jnp.tanh(c_st)
        hst_ref[pl.ds(k, 1), :] = h
        hrev_ref[pl.ds(kr, 1), :] = h

    feats = (lax.dot_general(hst_ref[...], a2, dn, preferred_element_type=f32)
             + lax.dot_general(hrev_ref[...], b2, dn, preferred_element_type=f32)
             + bt)
    ft_t = feats.T

    lane_t = lax.broadcasted_iota(jnp.int32, (1, T), 1)
    lane2 = lax.broadcasted_iota(jnp.int32, (T, T), 1)
    sub2 = lax.broadcasted_iota(jnp.int32, (T, T), 0)
    trans_t = trans.T
    z55 = jnp.zeros((T, T), f32)
    ftreps = [ft_t[:, t:t + 1] + z55 for t in range(0, S, 2)]
    frows = [feats[t:t + 1, :] for t in range(1, S, 2)]

    fv_row = jnp.where(lane_t == START, 0.0, NEG)
    fv_col = None
    for t in range(S):
        if t % 2 == 0:
            nvar = trans + fv_row
            best = jnp.max(nvar, axis=1, keepdims=True)
            bp = jnp.min(jnp.where(nvar == best, lane2, T), axis=1,
                         keepdims=True)
            bpc_ref[:, t:t + 1] = bp
            fv_col = best + ftreps[t // 2]
        else:
            nvar = trans_t + fv_col
            best = jnp.max(nvar, axis=0, keepdims=True)
            bp = jnp.min(jnp.where(nvar == best, sub2, T), axis=0,
                         keepdims=True)
            bpr_ref[pl.ds(t, 1), :] = bp
            fv_row = best + frows[t // 2]

    cp_c = pltpu.make_async_copy(bpc_ref, sm_bpc, sems.at[0])
    cp_r = pltpu.make_async_copy(bpr_ref, sm_bpr, sems.at[1])
    cp_c.start()
    cp_r.start()

    terminal = fv_row + trans[STOP:STOP + 1, :]
    score_ref[...] = jnp.max(terminal, axis=1, keepdims=True)
    term_ref[...] = terminal
    cp_t = pltpu.make_async_copy(term_ref, sm_term, sems.at[2])
    cp_t.start()

    cp_c.wait()
    cp_r.wait()
    cp_t.wait()

    best_v = sm_term[0, 0]
    best_i = jnp.int32(0)
    for j in range(1, T):
        better = sm_term[0, j] > best_v
        best_v = jnp.where(better, sm_term[0, j], best_v)
        best_i = jnp.where(better, jnp.int32(j), best_i)

    iota_s = lax.broadcasted_iota(jnp.int32, (1, S), 1)
    cur = best_i
    path_row = jnp.where(iota_s == (S - 1), cur, 0)
    for k in range(S - 1):
        t = S - 1 - k
        if t % 2 == 0:
            prev = sm_bpc[cur, t]
        else:
            prev = sm_bpr[t, cur]
        path_row = jnp.where(iota_s == (t - 1), prev, path_row)
        cur = prev
    path_ref[...] = path_row


def kernel(sentence, embedding, w_ih_f, w_hh_f, b_ih_f, b_hh_f,
           w_ih_b, w_hh_b, b_ih_b, b_hh_b, w_h2t, b_h2t, transitions, h0, c0):
    S = sentence.shape[0]
    f32 = jnp.float32

    embs = embedding[sentence]

    packed = jnp.concatenate([
        w_ih_f, w_ih_b, w_hh_f, w_hh_b,
        (b_ih_f + b_hh_f).reshape(4, HID),
        (b_ih_b + b_hh_b).reshape(4, HID),
        w_h2t[:, :HID], w_h2t[:, HID:],
        jnp.pad(b_h2t.reshape(1, T), ((0, 0), (0, HID - T))),
        jnp.pad(transitions, ((0, 0), (0, HID - T))),
        h0.reshape(2, HID), c0.reshape(2, HID),
    ], axis=0)

    def _vmem_spec(shape):
        nd = len(shape)
        return pl.BlockSpec(shape, lambda *_, _nd=nd: (0,) * _nd)

    score, path = pl.pallas_call(
        _bilstm_crf_fused,
        out_shape=(jax.ShapeDtypeStruct((1, 1), f32),
                   jax.ShapeDtypeStruct((1, S), jnp.int32)),
        grid_spec=pltpu.PrefetchScalarGridSpec(
            num_scalar_prefetch=0,
            grid=(1,),
            in_specs=[_vmem_spec(embs.shape), _vmem_spec(packed.shape)],
            out_specs=[_vmem_spec((1, 1)), _vmem_spec((1, S))],
            scratch_shapes=[
                pltpu.VMEM((S, 512), f32),
                pltpu.VMEM((S, 512), f32),
                pltpu.VMEM((S, 2 * HID), f32),
                pltpu.VMEM((S, 2 * HID), f32),
                pltpu.VMEM((T, S), jnp.int32),
                pltpu.VMEM((S, T), jnp.int32),
                pltpu.VMEM((1, T), f32),
                pltpu.SMEM((T, S), jnp.int32),
                pltpu.SMEM((S, T), jnp.int32),
                pltpu.SMEM((1, T), f32),
                pltpu.SemaphoreType.DMA((3,)),
            ]),
        compiler_params=pltpu.CompilerParams(
            dimension_semantics=("arbitrary",)),
    )(embs, packed)
    return score[0, 0], path[0, :]
